# baseline (speedup 1.0000x reference)
import functools

import jax
import jax.numpy as jnp
from jax import lax
from jax.experimental import pallas as pl
from jax.experimental.pallas import tpu as pltpu
from jax.experimental.pallas import tpu_sc as plsc

N = 10000
E = 320000
D = 128

NC = 2
NS = 16
NW = NC * NS
EW = E // NW
K = 125
NCHUNK = EW // K
PH = NCHUNK // 2
KR = 128
NP = 10240
RPT = NP // NS

_mesh = plsc.VectorSubcoreMesh(core_axis_name="c", subcore_axis_name="s")


def _sc_agg_body(x_hbm, ei_hbm, out0_hbm, out1_hbm,
                 sbuf, dbuf, rows0, rows1, agg,
                 gsem0, gsem1, ssem0, ssem1):
    c = lax.axis_index("c")
    s = lax.axis_index("s")
    wid = s * NC + c

    zeros16 = jnp.zeros((16,), jnp.float32)

    def zfill(i, carry):
        for j in range(D // 16):
            rows0[i, pl.ds(j * 16, 16)] = zeros16
        return carry

    lax.fori_loop(0, KR, zfill, 0)

    gsem = (gsem0, gsem1)
    ssem = (ssem0, ssem1)
    rr = (rows0.at[pl.ds(0, K)], rows1.at[pl.ds(0, K)])

    for p in range(2):
        pltpu.sync_copy(ei_hbm.at[0].at[pl.ds(wid * NCHUNK + p * PH, PH)],
                        sbuf)
        pltpu.sync_copy(ei_hbm.at[1].at[pl.ds(wid * NCHUNK + p * PH, PH)],
                        dbuf)
        if p == 0:
            pltpu.async_copy(x_hbm.at[sbuf.at[1]], rr[1], gsem1)
            for t in range(RPT // KR):
                pltpu.async_copy(rows0, agg.at[pl.ds(s * RPT + t * KR, KR)],
                                 ssem0)
            for t in range(RPT // KR):
                pltpu.make_async_copy(
                    rows0, agg.at[pl.ds(s * RPT + t * KR, KR)],
                    ssem0).wait()
            pltpu.async_copy(x_hbm.at[sbuf.at[0]], rr[0], gsem0)
            plsc.subcore_barrier()
        else:
            pltpu.async_copy(x_hbm.at[sbuf.at[0]], rr[0], gsem0)
            pltpu.async_copy(x_hbm.at[sbuf.at[1]], rr[1], gsem1)

        def body(i, carry):
            for b in range(2):
                j = 2 * i + b
                pltpu.make_async_copy(x_hbm.at[sbuf.at[j]], rr[b],
                                      gsem[b]).wait()
                pltpu.async_copy(rr[b], agg.at[dbuf.at[j]], ssem[b],
                                 add=True)
                pltpu.make_async_copy(rr[b], agg.at[dbuf.at[j]],
                                      ssem[b]).wait()
                pltpu.async_copy(x_hbm.at[sbuf.at[j + 2]], rr[b], gsem[b])
            return carry

        lax.fori_loop(0, PH // 2 - 1, body, 0)
        for b in range(2):
            j = PH - 2 + b
            pltpu.make_async_copy(x_hbm.at[sbuf.at[j]], rr[b],
                                  gsem[b]).wait()
            pltpu.async_copy(rr[b], agg.at[dbuf.at[j]], ssem[b], add=True)
            pltpu.make_async_copy(rr[b], agg.at[dbuf.at[j]], ssem[b]).wait()
    plsc.subcore_barrier()

    @pl.when(c == 0)
    def _():
        pltpu.sync_copy(agg.at[pl.ds(s * RPT, RPT)],
                        out0_hbm.at[pl.ds(s * RPT, RPT)])

    @pl.when(c == 1)
    def _():
        pltpu.sync_copy(agg.at[pl.ds(s * RPT, RPT)],
                        out1_hbm.at[pl.ds(s * RPT, RPT)])


_sc_agg = functools.partial(
    pl.kernel,
    out_type=(jax.ShapeDtypeStruct((NP, D), jnp.float32),
              jax.ShapeDtypeStruct((NP, D), jnp.float32)),
    mesh=_mesh,
    scratch_types=[
        pltpu.VMEM((PH, K), jnp.int32),
        pltpu.VMEM((PH, K), jnp.int32),
        pltpu.VMEM((KR, D), jnp.float32),
        pltpu.VMEM((KR, D), jnp.float32),
        pltpu.VMEM_SHARED((NP, D), jnp.float32),
        pltpu.SemaphoreType.DMA,
        pltpu.SemaphoreType.DMA,
        pltpu.SemaphoreType.DMA,
        pltpu.SemaphoreType.DMA,
    ],
)(_sc_agg_body)


BM = 1000


_DN = (((1,), (1,)), ((), ()))


def _tc_body(relu, a0_ref, a1_ref, x_ref, wr_ref, b_ref, wt_ref, o_ref):
    agg = a0_ref[...] + a1_ref[...]
    z = (lax.dot_general(agg, wr_ref[...], _DN,
                         preferred_element_type=jnp.float32)
         + b_ref[...]
         + lax.dot_general(x_ref[...], wt_ref[...], _DN,
                           preferred_element_type=jnp.float32))
    o_ref[...] = jnp.maximum(z, 0.0) if relu else z


def _tc_layer(a0, a1, x, w_rel_t, b_rel, w_root_t, relu):
    return pl.pallas_call(
        functools.partial(_tc_body, relu),
        grid=(N // BM,),
        in_specs=[
            pl.BlockSpec((BM, D), lambda i: (i, 0)),
            pl.BlockSpec((BM, D), lambda i: (i, 0)),
            pl.BlockSpec((BM, D), lambda i: (i, 0)),
            pl.BlockSpec((D, D), lambda i: (0, 0)),
            pl.BlockSpec((1, D), lambda i: (0, 0)),
            pl.BlockSpec((D, D), lambda i: (0, 0)),
        ],
        out_specs=pl.BlockSpec((BM, D), lambda i: (i, 0)),
        out_shape=jax.ShapeDtypeStruct((N, D), jnp.float32),
    )(a0, a1, x, w_rel_t, b_rel, w_root_t)


@jax.jit
def kernel(x, edge_index, W_rel1, b_rel1, W_root1, W_rel2, b_rel2, W_root2):
    ei = edge_index.reshape(2, E // K, K)
    a10, a11 = _sc_agg(x, ei)
    h = _tc_layer(a10, a11, x, W_rel1, b_rel1.reshape(1, D), W_root1, True)
    a20, a21 = _sc_agg(h, ei)
    return _tc_layer(a20, a21, h, W_rel2, b_rel2.reshape(1, D), W_root2,
                     False)


# speedup vs baseline: 1.0348x; 1.0348x over previous
import functools

import jax
import jax.numpy as jnp
from jax import lax
from jax.experimental import pallas as pl
from jax.experimental.pallas import tpu as pltpu
from jax.experimental.pallas import tpu_sc as plsc

N = 10000
E = 320000
D = 128

NC = 2
NS = 16
NW = NC * NS
EW = E // NW
K = 125
NCHUNK = EW // K
PH = NCHUNK // 2
KR = 128
NP = 10240
RPT = NP // NS

_mesh = plsc.VectorSubcoreMesh(core_axis_name="c", subcore_axis_name="s")


def _sc_agg_body(x_hbm, ei_hbm, out0_hbm, out1_hbm,
                 sbuf, dbuf, rows0, rows1, agg,
                 gsem0, gsem1, ssem0, ssem1):
    c = lax.axis_index("c")
    s = lax.axis_index("s")
    wid = s * NC + c

    zeros16 = jnp.zeros((16,), jnp.float32)

    def zfill(i, carry):
        for j in range(D // 16):
            rows0[i, pl.ds(j * 16, 16)] = zeros16
        return carry

    lax.fori_loop(0, KR, zfill, 0)

    gsem = (gsem0, gsem1)
    ssem = (ssem0, ssem1)
    rr = (rows0.at[pl.ds(0, K)], rows1.at[pl.ds(0, K)])

    for p in range(2):
        pltpu.sync_copy(ei_hbm.at[0].at[pl.ds(wid * NCHUNK + p * PH, PH)],
                        sbuf)
        pltpu.sync_copy(ei_hbm.at[1].at[pl.ds(wid * NCHUNK + p * PH, PH)],
                        dbuf)
        if p == 0:
            pltpu.async_copy(x_hbm.at[sbuf.at[1]], rr[1], gsem1)
            for t in range(RPT // KR):
                pltpu.async_copy(rows0, agg.at[pl.ds(s * RPT + t * KR, KR)],
                                 ssem0)
            for t in range(RPT // KR):
                pltpu.make_async_copy(
                    rows0, agg.at[pl.ds(s * RPT + t * KR, KR)],
                    ssem0).wait()
            pltpu.async_copy(x_hbm.at[sbuf.at[0]], rr[0], gsem0)
            plsc.subcore_barrier()
        else:
            pltpu.async_copy(x_hbm.at[sbuf.at[0]], rr[0], gsem0)
            pltpu.async_copy(x_hbm.at[sbuf.at[1]], rr[1], gsem1)

        def body(i, carry):
            for b in range(2):
                j = 2 * i + b
                pltpu.make_async_copy(x_hbm.at[sbuf.at[j]], rr[b],
                                      gsem[b]).wait()
                pltpu.async_copy(rr[b], agg.at[dbuf.at[j]], ssem[b],
                                 add=True)
                pltpu.make_async_copy(rr[b], agg.at[dbuf.at[j]],
                                      ssem[b]).wait()
                pltpu.async_copy(x_hbm.at[sbuf.at[j + 2]], rr[b], gsem[b])
            return carry

        lax.fori_loop(0, PH // 2 - 1, body, 0)
        for b in range(2):
            j = PH - 2 + b
            pltpu.make_async_copy(x_hbm.at[sbuf.at[j]], rr[b],
                                  gsem[b]).wait()
            pltpu.async_copy(rr[b], agg.at[dbuf.at[j]], ssem[b], add=True)
            pltpu.make_async_copy(rr[b], agg.at[dbuf.at[j]], ssem[b]).wait()
    plsc.subcore_barrier()

    @pl.when(c == 0)
    def _():
        pltpu.sync_copy(agg.at[pl.ds(s * RPT, RPT)],
                        out0_hbm.at[pl.ds(s * RPT, RPT)])

    @pl.when(c == 1)
    def _():
        pltpu.sync_copy(agg.at[pl.ds(s * RPT, RPT)],
                        out1_hbm.at[pl.ds(s * RPT, RPT)])


_sc_agg = functools.partial(
    pl.kernel,
    out_type=(jax.ShapeDtypeStruct((NP, D), jnp.float32),
              jax.ShapeDtypeStruct((NP, D), jnp.float32)),
    mesh=_mesh,
    scratch_types=[
        pltpu.VMEM((PH, K), jnp.int32),
        pltpu.VMEM((PH, K), jnp.int32),
        pltpu.VMEM((KR, D), jnp.float32),
        pltpu.VMEM((KR, D), jnp.float32),
        pltpu.VMEM_SHARED((NP, D), jnp.float32),
        pltpu.SemaphoreType.DMA,
        pltpu.SemaphoreType.DMA,
        pltpu.SemaphoreType.DMA,
        pltpu.SemaphoreType.DMA,
    ],
)(_sc_agg_body)


BM = 5000


_DN = (((1,), (1,)), ((), ()))


def _tc_body(relu, a0_ref, a1_ref, x_ref, wr_ref, b_ref, wt_ref, o_ref):
    agg = a0_ref[...] + a1_ref[...]
    z = (lax.dot_general(agg, wr_ref[...], _DN,
                         preferred_element_type=jnp.float32)
         + b_ref[...]
         + lax.dot_general(x_ref[...], wt_ref[...], _DN,
                           preferred_element_type=jnp.float32))
    o_ref[...] = jnp.maximum(z, 0.0) if relu else z


def _tc_layer(a0, a1, x, w_rel_t, b_rel, w_root_t, relu):
    return pl.pallas_call(
        functools.partial(_tc_body, relu),
        grid=(N // BM,),
        in_specs=[
            pl.BlockSpec((BM, D), lambda i: (i, 0)),
            pl.BlockSpec((BM, D), lambda i: (i, 0)),
            pl.BlockSpec((BM, D), lambda i: (i, 0)),
            pl.BlockSpec((D, D), lambda i: (0, 0)),
            pl.BlockSpec((1, D), lambda i: (0, 0)),
            pl.BlockSpec((D, D), lambda i: (0, 0)),
        ],
        out_specs=pl.BlockSpec((BM, D), lambda i: (i, 0)),
        out_shape=jax.ShapeDtypeStruct((N, D), jnp.float32),
    )(a0, a1, x, w_rel_t, b_rel, w_root_t)


@jax.jit
def kernel(x, edge_index, W_rel1, b_rel1, W_root1, W_rel2, b_rel2, W_root2):
    ei = edge_index.reshape(2, E // K, K)
    a10, a11 = _sc_agg(x, ei)
    h = _tc_layer(a10, a11, x, W_rel1, b_rel1.reshape(1, D), W_root1, True)
    a20, a21 = _sc_agg(h, ei)
    return _tc_layer(a20, a21, h, W_rel2, b_rel2.reshape(1, D), W_root2,
                     False)
